# Initial kernel scaffold; baseline (speedup 1.0000x reference)
import jax
import jax.numpy as jnp
from jax.experimental import pallas as pl


def kernel(input_ids, embed_table):
    raise NotImplementedError("write your pallas kernel here")


# speedup vs baseline: 1.6216x; 1.6216x over previous
import functools

import jax
import jax.numpy as jnp
from jax import lax
from jax.experimental import pallas as pl
from jax.experimental.pallas import tpu as pltpu
from jax.experimental.pallas import tpu_sc as plsc

_D = 4096
_N = 8192
_NC = 2
_NS = 16
_NW = _NC * _NS
_PER_W = _N // _NW
_C = 16
_NCHUNK = _PER_W // _C

_mesh = plsc.VectorSubcoreMesh(
    core_axis_name="c", subcore_axis_name="s",
    num_cores=_NC, num_subcores=_NS)


@functools.partial(
    pl.kernel,
    out_type=jax.ShapeDtypeStruct((_N, _D), jnp.float32),
    mesh=_mesh,
    scratch_types=[
        pltpu.VMEM((_PER_W,), jnp.int32),
        pltpu.VMEM((_C, _D), jnp.float32),
        pltpu.SemaphoreType.DMA,
    ],
)
def _embed_gather(ids_hbm, table_hbm, out_hbm, idx_v, buf, gsem):
    wid = lax.axis_index("s") * _NC + lax.axis_index("c")
    base = wid * _PER_W
    pltpu.sync_copy(ids_hbm.at[pl.ds(base, _PER_W)], idx_v)

    @pl.loop(0, _NCHUNK)
    def _chunk(g):
        row = pl.multiple_of(g * _C, 8)
        pltpu.async_copy(
            table_hbm.at[idx_v.at[pl.ds(row, _C)]], buf, gsem).wait()
        pltpu.sync_copy(buf, out_hbm.at[pl.ds(base + row, _C)])


def kernel(input_ids, embed_table):
    ids = input_ids.reshape(-1).astype(jnp.int32)
    out = _embed_gather(ids, embed_table)
    return out.reshape(input_ids.shape + (embed_table.shape[1],))


# speedup vs baseline: 1.6827x; 1.0377x over previous
import functools

import jax
import jax.numpy as jnp
from jax import lax
from jax.experimental import pallas as pl
from jax.experimental.pallas import tpu as pltpu
from jax.experimental.pallas import tpu_sc as plsc

_D = 4096
_N = 8192
_NC = 2
_NS = 16
_NW = _NC * _NS
_PER_W = _N // _NW
_C = 8
_NBUF = 2
_NCHUNK = _PER_W // _C
_NPAIR = _NCHUNK // _NBUF

_mesh = plsc.VectorSubcoreMesh(
    core_axis_name="c", subcore_axis_name="s",
    num_cores=_NC, num_subcores=_NS)


@functools.partial(
    pl.kernel,
    out_type=jax.ShapeDtypeStruct((_N, _D), jnp.float32),
    mesh=_mesh,
    scratch_types=[
        pltpu.VMEM((_PER_W,), jnp.int32),
        pltpu.VMEM((_NBUF, _C, _D), jnp.float32),
        pltpu.SemaphoreType.DMA,
        pltpu.SemaphoreType.DMA,
        pltpu.SemaphoreType.DMA,
        pltpu.SemaphoreType.DMA,
    ],
)
def _embed_gather(ids_hbm, table_hbm, out_hbm, idx_v, buf, g0, g1, s0, s1):
    wid = lax.axis_index("s") * _NC + lax.axis_index("c")
    base = wid * _PER_W
    gsem = (g0, g1)
    ssem = (s0, s1)
    pltpu.sync_copy(ids_hbm.at[pl.ds(base, _PER_W)], idx_v)

    def gather_start(c, b):
        row = pl.multiple_of(c * _C, 8)
        pltpu.async_copy(
            table_hbm.at[idx_v.at[pl.ds(row, _C)]], buf.at[b], gsem[b])

    def gather_wait(b):
        pltpu.make_async_copy(
            table_hbm.at[pl.ds(0, _C)], buf.at[b], gsem[b]).wait()

    def scatter_start(c, b):
        row = pl.multiple_of(c * _C, 8)
        pltpu.async_copy(
            buf.at[b], out_hbm.at[pl.ds(base + row, _C)], ssem[b])

    def scatter_wait(b):
        pltpu.make_async_copy(
            buf.at[b], out_hbm.at[pl.ds(base, _C)], ssem[b]).wait()

    for b in range(_NBUF):
        gather_start(b, b)

    @pl.loop(0, _NPAIR - 1)
    def _pair(p):
        c0 = p * _NBUF
        for b in range(_NBUF):
            gather_wait(b)
            scatter_start(c0 + b, b)
        for b in range(_NBUF):
            scatter_wait(b)
            gather_start(c0 + b + _NBUF, b)

    for b in range(_NBUF):
        gather_wait(b)
        scatter_start(_NCHUNK - _NBUF + b, b)
    for b in range(_NBUF):
        scatter_wait(b)


def kernel(input_ids, embed_table):
    ids = input_ids.reshape(-1).astype(jnp.int32)
    out = _embed_gather(ids, embed_table)
    return out.reshape(input_ids.shape + (embed_table.shape[1],))
